# baseline (speedup 1.0000x reference)
import functools

import jax
import jax.numpy as jnp
from jax import lax
from jax.experimental import pallas as pl
from jax.experimental.pallas import tpu as pltpu
from jax.experimental.pallas import tpu_sc as plsc

NC = 2
NS = 16
NW = NC * NS
LANES = 16
CHUNK = 128
CH_C = 128


GRP = 8


def _temb_body(emb_ref, w_ref, out_ref):
    out_ref[0] = jnp.dot(emb_ref[...], w_ref[0],
                         preferred_element_type=jnp.float32)


def _self_body(x_ref, w_ref, out_ref):
    out_ref[0] = jnp.dot(x_ref[0], w_ref[...],
                         preferred_element_type=jnp.float32)


def _make_scatter_kernel(n_nodes, d, n_chunks, chunks_per_rel, n_rel):
    per_tile = n_chunks // NW
    mesh = plsc.VectorSubcoreMesh(core_axis_name="c", subcore_axis_name="s")
    full_blocks = n_nodes // CHUNK
    tail_rows = n_nodes - full_blocks * CHUNK
    init_iters = -(-full_blocks // NS)
    d8 = d // LANES

    @functools.partial(
        pl.kernel,
        mesh=mesh,
        out_type=jax.ShapeDtypeStruct((NC, n_nodes, d), jnp.float32),
        scratch_types=[
            pltpu.VMEM((GRP, CHUNK), jnp.int32),
            pltpu.VMEM((GRP, CHUNK), jnp.int32),
            pltpu.VMEM((GRP, CHUNK), jnp.float32),
            pltpu.VMEM((CHUNK, d), jnp.float32),
            pltpu.VMEM_SHARED((n_nodes, d), jnp.float32),
            pltpu.SemaphoreType.DMA,
        ],
    )
    def scatter_kernel(rows_hbm, cols_hbm, vals_hbm, temb_hbm, out_hbm,
                       rg, cadj, vb, gbuf, acc, sem):
        c = lax.axis_index("c")
        s = lax.axis_index("s")
        wid = s * NC + c

        zvec = jnp.zeros((LANES,), jnp.float32)

        def zrow(r, carry):
            for k8 in range(d8):
                gbuf[r, pl.ds(k8 * LANES, LANES)] = zvec
            return carry

        lax.fori_loop(0, CHUNK, zrow, 0)
        for j in range(init_iters):
            blk = s + j * NS

            @pl.when(blk < full_blocks)
            def _():
                pltpu.async_copy(gbuf, acc.at[pl.ds(blk * CHUNK, CHUNK)],
                                 sem)

        if tail_rows:
            @pl.when(s == NS - 1)
            def _():
                pltpu.async_copy(
                    gbuf.at[pl.ds(0, tail_rows)],
                    acc.at[pl.ds(full_blocks * CHUNK, tail_rows)], sem)

        for j in range(init_iters):
            blk = s + j * NS

            @pl.when(blk < full_blocks)
            def _():
                pltpu.make_async_copy(
                    gbuf, acc.at[pl.ds(blk * CHUNK, CHUNK)], sem).wait()

        if tail_rows:
            @pl.when(s == NS - 1)
            def _():
                pltpu.make_async_copy(
                    gbuf.at[pl.ds(0, tail_rows)],
                    acc.at[pl.ds(full_blocks * CHUNK, tail_rows)], sem).wait()

        plsc.subcore_barrier()

        first = wid * per_tile

        def grp_body(g, carry):
            base = first + g * GRP
            pltpu.sync_copy(rows_hbm.at[pl.ds(base, GRP)], rg)
            pltpu.sync_copy(cols_hbm.at[pl.ds(base, GRP)], cadj)
            pltpu.sync_copy(vals_hbm.at[pl.ds(base, GRP)], vb)
            for jj in range(GRP):
                off = (jnp.minimum((base + jj) // chunks_per_rel, n_rel - 1)
                       * n_nodes)
                for k8 in range(CHUNK // LANES):
                    sl = pl.ds(k8 * LANES, LANES)
                    cadj[jj, sl] = cadj[jj, sl] + off
            for jj in range(GRP):
                pltpu.async_copy(temb_hbm.at[cadj.at[jj]], gbuf, sem).wait()

                def scale(gq, inner):
                    vvec = vb[jj, pl.ds(gq * LANES, LANES)]
                    for lane in range(LANES):
                        v = vvec[lane]
                        row = gq * LANES + lane
                        for k8 in range(d8):
                            sl = pl.ds(k8 * LANES, LANES)
                            gbuf[row, sl] = gbuf[row, sl] * v
                    return inner

                lax.fori_loop(0, CHUNK // LANES, scale, 0)
                pltpu.sync_copy(gbuf, acc.at[rg.at[jj]], add=True)
            return carry

        lax.fori_loop(0, per_tile // GRP, grp_body, 0)
        plsc.subcore_barrier()

        for j in range(init_iters):
            blk = s + j * NS

            @pl.when(blk < full_blocks)
            def _():
                pltpu.async_copy(
                    acc.at[pl.ds(blk * CHUNK, CHUNK)],
                    out_hbm.at[c, pl.ds(blk * CHUNK, CHUNK)], sem)

        if tail_rows:
            @pl.when(s == NS - 1)
            def _():
                pltpu.async_copy(
                    acc.at[pl.ds(full_blocks * CHUNK, tail_rows)],
                    out_hbm.at[c, pl.ds(full_blocks * CHUNK, tail_rows)], sem)

        for j in range(init_iters):
            blk = s + j * NS

            @pl.when(blk < full_blocks)
            def _():
                pltpu.make_async_copy(
                    acc.at[pl.ds(blk * CHUNK, CHUNK)],
                    out_hbm.at[c, pl.ds(blk * CHUNK, CHUNK)], sem).wait()

        if tail_rows:
            @pl.when(s == NS - 1)
            def _():
                pltpu.make_async_copy(
                    acc.at[pl.ds(full_blocks * CHUNK, tail_rows)],
                    out_hbm.at[c, pl.ds(full_blocks * CHUNK, tail_rows)],
                    sem).wait()

    return scatter_kernel


def _make_combine_kernel(n_nodes, d, total_rows):
    rows_per_tile = total_rows // NW
    n_sub = rows_per_tile // CH_C
    d8 = d // LANES
    mesh = plsc.VectorSubcoreMesh(core_axis_name="c", subcore_axis_name="s")

    @functools.partial(
        pl.kernel,
        mesh=mesh,
        out_type=jax.ShapeDtypeStruct((total_rows, d), jnp.float32),
        scratch_types=[
            pltpu.VMEM((CH_C,), jnp.int32),
            pltpu.VMEM((CH_C,), jnp.int32),
            pltpu.VMEM((CH_C, d), jnp.float32),
            pltpu.VMEM((CH_C, d), jnp.float32),
            pltpu.VMEM((CH_C, d), jnp.float32),
            pltpu.VMEM((CH_C, d), jnp.float32),
            pltpu.SemaphoreType.DMA,
        ],
    )
    def combine_kernel(acc_hbm, idx_hbm, self_hbm, out_hbm,
                       ibuf, ibufn, g0, g1, sbuf, obuf, sem):
        c = lax.axis_index("c")
        s = lax.axis_index("s")
        wid = s * NC + c
        base = wid * rows_per_tile

        for t in range(n_sub):
            b0 = base + t * CH_C
            pltpu.sync_copy(idx_hbm.at[pl.ds(b0, CH_C)], ibuf)
            for k8 in range(CH_C // LANES):
                sl = pl.ds(k8 * LANES, LANES)
                ibufn[sl] = ibuf[sl] + n_nodes
            cp0 = pltpu.async_copy(acc_hbm.at[ibuf], g0, sem)
            cp1 = pltpu.async_copy(acc_hbm.at[ibufn], g1, sem)
            pltpu.sync_copy(self_hbm.at[pl.ds(b0, CH_C)], sbuf)
            cp0.wait()
            cp1.wait()

            def srow(k, carry):
                for k8 in range(d8):
                    sl = pl.ds(k8 * LANES, LANES)
                    x = g0[k, sl] + g1[k, sl] + sbuf[k, sl]
                    obuf[k, sl] = 1.0 / (1.0 + jnp.exp(-x))
                return carry

            lax.fori_loop(0, CH_C, srow, 0)
            pltpu.sync_copy(obuf, out_hbm.at[pl.ds(b0, CH_C)])

    return combine_kernel


def kernel(embeddings, head_idx, head_e, tail_idx, tail_e, adj_indices,
           adj_values, relation_kernel, self_kernel):
    n_nodes, d = embeddings.shape
    n_rel, _, n_edges = adj_indices.shape
    batch = head_e.shape[0]

    bn = 1000
    temb = pl.pallas_call(
        _temb_body,
        grid=(n_rel, n_nodes // bn),
        in_specs=[
            pl.BlockSpec((bn, d), lambda r, n: (n, 0)),
            pl.BlockSpec((1, d, d), lambda r, n: (r, 0, 0)),
        ],
        out_specs=pl.BlockSpec((1, bn, d), lambda r, n: (r, n, 0)),
        out_shape=jax.ShapeDtypeStruct((n_rel, n_nodes, d), jnp.float32),
    )(embeddings, relation_kernel)
    temb_flat = temb.reshape(n_rel * n_nodes, d)

    x = jnp.stack([head_e, tail_e])
    bm = 1024
    sout = pl.pallas_call(
        _self_body,
        grid=(2, batch // bm),
        in_specs=[
            pl.BlockSpec((1, bm, d), lambda i, m: (i, m, 0)),
            pl.BlockSpec((d, d), lambda i, m: (0, 0)),
        ],
        out_specs=pl.BlockSpec((1, bm, d), lambda i, m: (i, m, 0)),
        out_shape=jax.ShapeDtypeStruct((2, batch, d), jnp.float32),
    )(x, self_kernel)
    self_flat = sout.reshape(2 * batch, d)

    total_edges = n_rel * n_edges
    chunks_per_rel = n_edges // CHUNK
    per_tile = -(-(-(-total_edges // CHUNK)) // NW)
    per_tile = -(-per_tile // GRP) * GRP
    n_chunks = per_tile * NW
    pad = n_chunks * CHUNK - total_edges
    rows2d = jnp.pad(adj_indices[:, 0, :].astype(jnp.int32).reshape(-1),
                     (0, pad)).reshape(n_chunks, CHUNK)
    cols2d = jnp.pad(adj_indices[:, 1, :].astype(jnp.int32).reshape(-1),
                     (0, pad)).reshape(n_chunks, CHUNK)
    vals2d = jnp.pad(adj_values.reshape(-1), (0, pad)).reshape(n_chunks, CHUNK)

    scatter = _make_scatter_kernel(n_nodes, d, n_chunks, chunks_per_rel, n_rel)
    acc2 = scatter(rows2d, cols2d, vals2d, temb_flat)
    acc_flat = acc2.reshape(NC * n_nodes, d)

    idx_all = jnp.concatenate([head_idx.astype(jnp.int32),
                               tail_idx.astype(jnp.int32)])
    combine = _make_combine_kernel(n_nodes, d, 2 * batch)
    out_all = combine(acc_flat, idx_all, self_flat)

    return (out_all[:batch], out_all[batch:])


# speedup vs baseline: 1.4224x; 1.4224x over previous
import functools

import jax
import jax.numpy as jnp
from jax import lax
from jax.experimental import pallas as pl
from jax.experimental.pallas import tpu as pltpu
from jax.experimental.pallas import tpu_sc as plsc

NC = 2
NS = 16
NW = NC * NS
LANES = 16
CHUNK = 128


def _temb_body(emb_ref, w_ref, out_ref):
    out_ref[0] = jnp.dot(emb_ref[...], w_ref[0],
                         preferred_element_type=jnp.float32)


def _self_body(x_ref, w_ref, out_ref):
    out_ref[0] = jnp.dot(x_ref[0], w_ref[...],
                         preferred_element_type=jnp.float32)


def _make_scatter_kernel(n_nodes, d, n_chunks, chunks_per_rel):
    base_cnt = n_chunks // NW
    extra = n_chunks - base_cnt * NW
    mesh = plsc.VectorSubcoreMesh(core_axis_name="c", subcore_axis_name="s")
    full_blocks = n_nodes // CHUNK
    tail_rows = n_nodes - full_blocks * CHUNK
    init_iters = -(-full_blocks // NS)
    d8 = d // LANES

    @functools.partial(
        pl.kernel,
        mesh=mesh,
        out_type=jax.ShapeDtypeStruct((NC, n_nodes, d), jnp.float32),
        scratch_types=[
            pltpu.VMEM((1, CHUNK), jnp.int32),
            pltpu.VMEM((CHUNK,), jnp.int32),
            pltpu.VMEM((CHUNK,), jnp.int32),
            pltpu.VMEM((CHUNK,), jnp.float32),
            pltpu.VMEM((CHUNK, d), jnp.float32),
            pltpu.VMEM_SHARED((n_nodes, d), jnp.float32),
            pltpu.SemaphoreType.DMA,
        ],
    )
    def scatter_kernel(rows_hbm, cols_hbm, vals_hbm, temb_hbm, out_hbm,
                       ridx, craw, cadj, vbuf, gbuf, acc, sem):
        c = lax.axis_index("c")
        s = lax.axis_index("s")
        wid = s * NC + c

        zvec = jnp.zeros((LANES,), jnp.float32)

        def zrow(r, carry):
            for k8 in range(d8):
                gbuf[r, pl.ds(k8 * LANES, LANES)] = zvec
            return carry

        lax.fori_loop(0, CHUNK, zrow, 0)
        for j in range(init_iters):
            blk = s + j * NS

            @pl.when(blk < full_blocks)
            def _():
                pltpu.sync_copy(gbuf, acc.at[pl.ds(blk * CHUNK, CHUNK)])

        if tail_rows:
            @pl.when(s == NS - 1)
            def _():
                pltpu.sync_copy(
                    gbuf.at[pl.ds(0, tail_rows)],
                    acc.at[pl.ds(full_blocks * CHUNK, tail_rows)])

        plsc.subcore_barrier()

        first = wid * base_cnt + jnp.minimum(wid, extra)
        cnt = base_cnt + jnp.where(wid < extra, 1, 0)

        def body(j, carry):
            ch = first + j
            pltpu.sync_copy(rows_hbm.at[ch], ridx.at[0])
            pltpu.sync_copy(cols_hbm.at[ch], craw)
            pltpu.sync_copy(vals_hbm.at[ch], vbuf)
            rel_off = (ch // chunks_per_rel) * n_nodes
            for k8 in range(CHUNK // LANES):
                cadj[pl.ds(k8 * LANES, LANES)] = (
                    craw[pl.ds(k8 * LANES, LANES)] + rel_off)
            pltpu.async_copy(temb_hbm.at[cadj], gbuf, sem).wait()

            def scale(g, inner):
                vvec = vbuf[pl.ds(g * LANES, LANES)]
                for lane in range(LANES):
                    v = vvec[lane]
                    row = g * LANES + lane
                    for k8 in range(d8):
                        sl = pl.ds(k8 * LANES, LANES)
                        gbuf[row, sl] = gbuf[row, sl] * v
                return inner

            lax.fori_loop(0, CHUNK // LANES, scale, 0)
            pltpu.sync_copy(gbuf, acc.at[ridx.at[0]], add=True)
            return carry

        lax.fori_loop(0, cnt, body, 0)
        plsc.subcore_barrier()

        for j in range(init_iters):
            blk = s + j * NS

            @pl.when(blk < full_blocks)
            def _():
                pltpu.sync_copy(
                    acc.at[pl.ds(blk * CHUNK, CHUNK)],
                    out_hbm.at[c, pl.ds(blk * CHUNK, CHUNK)])

        if tail_rows:
            @pl.when(s == NS - 1)
            def _():
                pltpu.sync_copy(
                    acc.at[pl.ds(full_blocks * CHUNK, tail_rows)],
                    out_hbm.at[c, pl.ds(full_blocks * CHUNK, tail_rows)])

    return scatter_kernel


def _make_combine_kernel(n_nodes, d, total_rows):
    rows_per_tile = total_rows // NW
    n_sub = rows_per_tile // CHUNK
    d8 = d // LANES
    mesh = plsc.VectorSubcoreMesh(core_axis_name="c", subcore_axis_name="s")

    @functools.partial(
        pl.kernel,
        mesh=mesh,
        out_type=jax.ShapeDtypeStruct((total_rows, d), jnp.float32),
        scratch_types=[
            pltpu.VMEM((CHUNK,), jnp.int32),
            pltpu.VMEM((CHUNK,), jnp.int32),
            pltpu.VMEM((CHUNK, d), jnp.float32),
            pltpu.VMEM((CHUNK, d), jnp.float32),
            pltpu.VMEM((CHUNK, d), jnp.float32),
            pltpu.VMEM((CHUNK, d), jnp.float32),
            pltpu.SemaphoreType.DMA,
        ],
    )
    def combine_kernel(acc_hbm, idx_hbm, self_hbm, out_hbm,
                       ibuf, ibufn, g0, g1, sbuf, obuf, sem):
        c = lax.axis_index("c")
        s = lax.axis_index("s")
        wid = s * NC + c
        base = wid * rows_per_tile

        for t in range(n_sub):
            b0 = base + t * CHUNK
            pltpu.sync_copy(idx_hbm.at[pl.ds(b0, CHUNK)], ibuf)
            for k8 in range(CHUNK // LANES):
                sl = pl.ds(k8 * LANES, LANES)
                ibufn[sl] = ibuf[sl] + n_nodes
            cp0 = pltpu.async_copy(acc_hbm.at[ibuf], g0, sem)
            cp1 = pltpu.async_copy(acc_hbm.at[ibufn], g1, sem)
            pltpu.sync_copy(self_hbm.at[pl.ds(b0, CHUNK)], sbuf)
            cp0.wait()
            cp1.wait()

            def srow(k, carry):
                for k8 in range(d8):
                    sl = pl.ds(k8 * LANES, LANES)
                    x = g0[k, sl] + g1[k, sl] + sbuf[k, sl]
                    obuf[k, sl] = 1.0 / (1.0 + jnp.exp(-x))
                return carry

            lax.fori_loop(0, CHUNK, srow, 0)
            pltpu.sync_copy(obuf, out_hbm.at[pl.ds(b0, CHUNK)])

    return combine_kernel


def kernel(embeddings, head_idx, head_e, tail_idx, tail_e, adj_indices,
           adj_values, relation_kernel, self_kernel):
    n_nodes, d = embeddings.shape
    n_rel, _, n_edges = adj_indices.shape
    batch = head_e.shape[0]

    bn = 1000
    temb = pl.pallas_call(
        _temb_body,
        grid=(n_rel, n_nodes // bn),
        in_specs=[
            pl.BlockSpec((bn, d), lambda r, n: (n, 0)),
            pl.BlockSpec((1, d, d), lambda r, n: (r, 0, 0)),
        ],
        out_specs=pl.BlockSpec((1, bn, d), lambda r, n: (r, n, 0)),
        out_shape=jax.ShapeDtypeStruct((n_rel, n_nodes, d), jnp.float32),
    )(embeddings, relation_kernel)
    temb_flat = temb.reshape(n_rel * n_nodes, d)

    x = jnp.stack([head_e, tail_e])
    bm = 1024
    sout = pl.pallas_call(
        _self_body,
        grid=(2, batch // bm),
        in_specs=[
            pl.BlockSpec((1, bm, d), lambda i, m: (i, m, 0)),
            pl.BlockSpec((d, d), lambda i, m: (0, 0)),
        ],
        out_specs=pl.BlockSpec((1, bm, d), lambda i, m: (i, m, 0)),
        out_shape=jax.ShapeDtypeStruct((2, batch, d), jnp.float32),
    )(x, self_kernel)
    self_flat = sout.reshape(2 * batch, d)

    total_edges = n_rel * n_edges
    n_chunks = total_edges // CHUNK
    chunks_per_rel = n_edges // CHUNK
    rows2d = adj_indices[:, 0, :].astype(jnp.int32).reshape(n_chunks, CHUNK)
    cols2d = adj_indices[:, 1, :].astype(jnp.int32).reshape(n_chunks, CHUNK)
    vals2d = adj_values.reshape(n_chunks, CHUNK)

    scatter = _make_scatter_kernel(n_nodes, d, n_chunks, chunks_per_rel)
    acc2 = scatter(rows2d, cols2d, vals2d, temb_flat)
    acc_flat = acc2.reshape(NC * n_nodes, d)

    idx_all = jnp.concatenate([head_idx.astype(jnp.int32),
                               tail_idx.astype(jnp.int32)])
    combine = _make_combine_kernel(n_nodes, d, 2 * batch)
    out_all = combine(acc_flat, idx_all, self_flat)

    return (out_all[:batch], out_all[batch:])


# speedup vs baseline: 1.7134x; 1.2046x over previous
import functools

import jax
import jax.numpy as jnp
from jax import lax
from jax.experimental import pallas as pl
from jax.experimental.pallas import tpu as pltpu
from jax.experimental.pallas import tpu_sc as plsc

NC = 2
NS = 16
NW = NC * NS
LANES = 16
CHUNK = 128


def _temb_body(emb_ref, w_ref, out_ref):
    out_ref[0] = jnp.dot(emb_ref[...], w_ref[0],
                         preferred_element_type=jnp.float32)


def _self_body(x_ref, w_ref, out_ref):
    out_ref[0] = jnp.dot(x_ref[0], w_ref[...],
                         preferred_element_type=jnp.float32)


def _make_scatter_kernel(n_nodes, d, n_chunks, chunks_per_rel):
    base_cnt = n_chunks // NW
    extra = n_chunks - base_cnt * NW
    mesh = plsc.VectorSubcoreMesh(core_axis_name="c", subcore_axis_name="s")
    full_blocks = n_nodes // CHUNK
    tail_rows = n_nodes - full_blocks * CHUNK
    init_iters = -(-full_blocks // NS)
    d8 = d // LANES

    @functools.partial(
        pl.kernel,
        mesh=mesh,
        out_type=jax.ShapeDtypeStruct((NC, n_nodes, d), jnp.float32),
        scratch_types=[
            pltpu.VMEM((1, CHUNK), jnp.int32),
            pltpu.VMEM((CHUNK,), jnp.int32),
            pltpu.VMEM((CHUNK,), jnp.int32),
            pltpu.VMEM((CHUNK,), jnp.float32),
            pltpu.VMEM((CHUNK, d), jnp.float32),
            pltpu.VMEM_SHARED((n_nodes, d), jnp.float32),
            pltpu.SemaphoreType.DMA,
        ],
    )
    def scatter_kernel(rows_hbm, cols_hbm, vals_hbm, temb_hbm, out_hbm,
                       ridx, craw, cadj, vbuf, gbuf, acc, sem):
        c = lax.axis_index("c")
        s = lax.axis_index("s")
        wid = s * NC + c

        zvec = jnp.zeros((LANES,), jnp.float32)

        def zrow(r, carry):
            for k8 in range(d8):
                gbuf[r, pl.ds(k8 * LANES, LANES)] = zvec
            return carry

        lax.fori_loop(0, CHUNK, zrow, 0)
        for j in range(init_iters):
            blk = s + j * NS

            @pl.when(blk < full_blocks)
            def _():
                pltpu.sync_copy(gbuf, acc.at[pl.ds(blk * CHUNK, CHUNK)])

        if tail_rows:
            @pl.when(s == NS - 1)
            def _():
                pltpu.sync_copy(
                    gbuf.at[pl.ds(0, tail_rows)],
                    acc.at[pl.ds(full_blocks * CHUNK, tail_rows)])

        plsc.subcore_barrier()

        first = wid * base_cnt + jnp.minimum(wid, extra)
        cnt = base_cnt + jnp.where(wid < extra, 1, 0)

        def body(j, carry):
            ch = first + j
            pltpu.sync_copy(cols_hbm.at[ch], craw)
            rel_off = (ch // chunks_per_rel) * n_nodes
            for k8 in range(CHUNK // LANES):
                cadj[pl.ds(k8 * LANES, LANES)] = (
                    craw[pl.ds(k8 * LANES, LANES)] + rel_off)
            cp = pltpu.async_copy(temb_hbm.at[cadj], gbuf, sem)
            pltpu.sync_copy(rows_hbm.at[ch], ridx.at[0])
            pltpu.sync_copy(vals_hbm.at[ch], vbuf)
            cp.wait()

            def scale(g, inner):
                vvec = vbuf[pl.ds(g * LANES, LANES)]
                for lane in range(LANES):
                    v = vvec[lane]
                    row = g * LANES + lane
                    for k8 in range(d8):
                        sl = pl.ds(k8 * LANES, LANES)
                        gbuf[row, sl] = gbuf[row, sl] * v
                return inner

            lax.fori_loop(0, CHUNK // LANES, scale, 0)
            pltpu.sync_copy(gbuf, acc.at[ridx.at[0]], add=True)
            return carry

        lax.fori_loop(0, cnt, body, 0)
        plsc.subcore_barrier()

        for j in range(init_iters):
            blk = s + j * NS

            @pl.when(blk < full_blocks)
            def _():
                pltpu.sync_copy(
                    acc.at[pl.ds(blk * CHUNK, CHUNK)],
                    out_hbm.at[c, pl.ds(blk * CHUNK, CHUNK)])

        if tail_rows:
            @pl.when(s == NS - 1)
            def _():
                pltpu.sync_copy(
                    acc.at[pl.ds(full_blocks * CHUNK, tail_rows)],
                    out_hbm.at[c, pl.ds(full_blocks * CHUNK, tail_rows)])

    return scatter_kernel


def _make_combine_kernel(n_nodes, d, total_rows):
    rows_per_tile = total_rows // NW
    n_sub = rows_per_tile // CHUNK
    d8 = d // LANES
    mesh = plsc.VectorSubcoreMesh(core_axis_name="c", subcore_axis_name="s")

    @functools.partial(
        pl.kernel,
        mesh=mesh,
        out_type=jax.ShapeDtypeStruct((total_rows, d), jnp.float32),
        scratch_types=[
            pltpu.VMEM((CHUNK,), jnp.int32),
            pltpu.VMEM((CHUNK,), jnp.int32),
            pltpu.VMEM((CHUNK, d), jnp.float32),
            pltpu.VMEM((CHUNK, d), jnp.float32),
            pltpu.VMEM((CHUNK, d), jnp.float32),
            pltpu.VMEM((CHUNK, d), jnp.float32),
            pltpu.SemaphoreType.DMA,
        ],
    )
    def combine_kernel(acc_hbm, idx_hbm, self_hbm, out_hbm,
                       ibuf, ibufn, g0, g1, sbuf, obuf, sem):
        c = lax.axis_index("c")
        s = lax.axis_index("s")
        wid = s * NC + c
        base = wid * rows_per_tile

        for t in range(n_sub):
            b0 = base + t * CHUNK
            pltpu.sync_copy(idx_hbm.at[pl.ds(b0, CHUNK)], ibuf)
            for k8 in range(CHUNK // LANES):
                sl = pl.ds(k8 * LANES, LANES)
                ibufn[sl] = ibuf[sl] + n_nodes
            cp0 = pltpu.async_copy(acc_hbm.at[ibuf], g0, sem)
            cp1 = pltpu.async_copy(acc_hbm.at[ibufn], g1, sem)
            pltpu.sync_copy(self_hbm.at[pl.ds(b0, CHUNK)], sbuf)
            cp0.wait()
            cp1.wait()

            def srow(k, carry):
                for k8 in range(d8):
                    sl = pl.ds(k8 * LANES, LANES)
                    x = g0[k, sl] + g1[k, sl] + sbuf[k, sl]
                    obuf[k, sl] = 1.0 / (1.0 + jnp.exp(-x))
                return carry

            lax.fori_loop(0, CHUNK, srow, 0)
            pltpu.sync_copy(obuf, out_hbm.at[pl.ds(b0, CHUNK)])

    return combine_kernel


def kernel(embeddings, head_idx, head_e, tail_idx, tail_e, adj_indices,
           adj_values, relation_kernel, self_kernel):
    n_nodes, d = embeddings.shape
    n_rel, _, n_edges = adj_indices.shape
    batch = head_e.shape[0]

    bn = 1000
    temb = pl.pallas_call(
        _temb_body,
        grid=(n_rel, n_nodes // bn),
        in_specs=[
            pl.BlockSpec((bn, d), lambda r, n: (n, 0)),
            pl.BlockSpec((1, d, d), lambda r, n: (r, 0, 0)),
        ],
        out_specs=pl.BlockSpec((1, bn, d), lambda r, n: (r, n, 0)),
        out_shape=jax.ShapeDtypeStruct((n_rel, n_nodes, d), jnp.float32),
    )(embeddings, relation_kernel)
    temb_flat = temb.reshape(n_rel * n_nodes, d)

    x = jnp.stack([head_e, tail_e])
    bm = 1024
    sout = pl.pallas_call(
        _self_body,
        grid=(2, batch // bm),
        in_specs=[
            pl.BlockSpec((1, bm, d), lambda i, m: (i, m, 0)),
            pl.BlockSpec((d, d), lambda i, m: (0, 0)),
        ],
        out_specs=pl.BlockSpec((1, bm, d), lambda i, m: (i, m, 0)),
        out_shape=jax.ShapeDtypeStruct((2, batch, d), jnp.float32),
    )(x, self_kernel)
    self_flat = sout.reshape(2 * batch, d)

    total_edges = n_rel * n_edges
    n_chunks = total_edges // CHUNK
    chunks_per_rel = n_edges // CHUNK
    rows2d = adj_indices[:, 0, :].astype(jnp.int32).reshape(n_chunks, CHUNK)
    cols2d = adj_indices[:, 1, :].astype(jnp.int32).reshape(n_chunks, CHUNK)
    vals2d = adj_values.reshape(n_chunks, CHUNK)

    scatter = _make_scatter_kernel(n_nodes, d, n_chunks, chunks_per_rel)
    acc2 = scatter(rows2d, cols2d, vals2d, temb_flat)
    acc_flat = acc2.reshape(NC * n_nodes, d)

    idx_all = jnp.concatenate([head_idx.astype(jnp.int32),
                               tail_idx.astype(jnp.int32)])
    combine = _make_combine_kernel(n_nodes, d, 2 * batch)
    out_all = combine(acc_flat, idx_all, self_flat)

    return (out_all[:batch], out_all[batch:])


# speedup vs baseline: 1.9010x; 1.1095x over previous
import functools

import jax
import jax.numpy as jnp
from jax import lax
from jax.experimental import pallas as pl
from jax.experimental.pallas import tpu as pltpu
from jax.experimental.pallas import tpu_sc as plsc

NC = 2
NS = 16
NW = NC * NS
LANES = 16
CHUNK = 128


def _temb_body(emb_ref, w_ref, out_ref):
    out_ref[0] = jnp.dot(emb_ref[...], w_ref[0],
                         preferred_element_type=jnp.float32)


def _self_body(x_ref, w_ref, out_ref):
    out_ref[0] = jnp.dot(x_ref[0], w_ref[...],
                         preferred_element_type=jnp.float32)


def _make_scatter_kernel(n_nodes, d, n_chunks, chunks_per_rel):
    base_cnt = n_chunks // NW
    extra = n_chunks - base_cnt * NW
    mesh = plsc.VectorSubcoreMesh(core_axis_name="c", subcore_axis_name="s")
    full_blocks = n_nodes // CHUNK
    tail_rows = n_nodes - full_blocks * CHUNK
    init_iters = -(-full_blocks // NS)
    d8 = d // LANES

    @functools.partial(
        pl.kernel,
        mesh=mesh,
        out_type=jax.ShapeDtypeStruct((NC, n_nodes, d), jnp.float32),
        scratch_types=[
            pltpu.VMEM((1, CHUNK), jnp.int32),
            pltpu.VMEM((CHUNK,), jnp.int32),
            pltpu.VMEM((2, CHUNK), jnp.int32),
            pltpu.VMEM((CHUNK,), jnp.float32),
            pltpu.VMEM((CHUNK, d), jnp.float32),
            pltpu.VMEM_SHARED((n_nodes, d), jnp.float32),
            pltpu.SemaphoreType.DMA,
            pltpu.SemaphoreType.DMA,
        ],
    )
    def scatter_kernel(rows_hbm, cols_hbm, vals_hbm, temb_hbm, out_hbm,
                       ridx, craw, cadj, vbuf, gbuf, acc, sem, sem_s):
        c = lax.axis_index("c")
        s = lax.axis_index("s")
        wid = s * NC + c

        zvec = jnp.zeros((LANES,), jnp.float32)

        def zrow(r, carry):
            for k8 in range(d8):
                gbuf[r, pl.ds(k8 * LANES, LANES)] = zvec
            return carry

        lax.fori_loop(0, CHUNK, zrow, 0)
        for j in range(init_iters):
            blk = s + j * NS

            @pl.when(blk < full_blocks)
            def _():
                pltpu.sync_copy(gbuf, acc.at[pl.ds(blk * CHUNK, CHUNK)])

        if tail_rows:
            @pl.when(s == NS - 1)
            def _():
                pltpu.sync_copy(
                    gbuf.at[pl.ds(0, tail_rows)],
                    acc.at[pl.ds(full_blocks * CHUNK, tail_rows)])

        plsc.subcore_barrier()

        first = wid * base_cnt + jnp.minimum(wid, extra)
        cnt = base_cnt + jnp.where(wid < extra, 1, 0)

        def prep_cols(a, ch):
            pltpu.sync_copy(cols_hbm.at[ch], craw)
            rel_off = (ch // chunks_per_rel) * n_nodes
            for k8 in range(CHUNK // LANES):
                cadj[a, pl.ds(k8 * LANES, LANES)] = (
                    craw[pl.ds(k8 * LANES, LANES)] + rel_off)

        def run_chunk(a, ch, nxt):
            cp = pltpu.async_copy(temb_hbm.at[cadj.at[a]], gbuf, sem)
            pltpu.sync_copy(rows_hbm.at[ch], ridx.at[0])
            pltpu.sync_copy(vals_hbm.at[ch], vbuf)
            cp.wait()

            def scale(g, inner):
                vvec = vbuf[pl.ds(g * LANES, LANES)]
                for lane in range(LANES):
                    v = vvec[lane]
                    row = g * LANES + lane
                    for k8 in range(d8):
                        sl = pl.ds(k8 * LANES, LANES)
                        gbuf[row, sl] = gbuf[row, sl] * v
                return inner

            lax.fori_loop(0, CHUNK // LANES, scale, 0)
            sp = pltpu.async_copy(gbuf, acc.at[ridx.at[0]], sem_s, add=True)
            if nxt is not None:
                @pl.when(nxt < first + cnt)
                def _():
                    prep_cols(1 - a, nxt)
            sp.wait()

        prep_cols(0, first)

        def body(t, carry):
            ca = first + 2 * t
            run_chunk(0, ca, ca + 1)
            run_chunk(1, ca + 1, ca + 2)
            return carry

        lax.fori_loop(0, cnt // 2, body, 0)

        @pl.when(cnt % 2 == 1)
        def _():
            run_chunk(0, first + cnt - 1, None)
        plsc.subcore_barrier()

        for j in range(init_iters):
            blk = s + j * NS

            @pl.when(blk < full_blocks)
            def _():
                pltpu.sync_copy(
                    acc.at[pl.ds(blk * CHUNK, CHUNK)],
                    out_hbm.at[c, pl.ds(blk * CHUNK, CHUNK)])

        if tail_rows:
            @pl.when(s == NS - 1)
            def _():
                pltpu.sync_copy(
                    acc.at[pl.ds(full_blocks * CHUNK, tail_rows)],
                    out_hbm.at[c, pl.ds(full_blocks * CHUNK, tail_rows)])

    return scatter_kernel


def _make_combine_kernel(n_nodes, d, total_rows):
    rows_per_tile = total_rows // NW
    n_sub = rows_per_tile // CHUNK
    d8 = d // LANES
    mesh = plsc.VectorSubcoreMesh(core_axis_name="c", subcore_axis_name="s")

    @functools.partial(
        pl.kernel,
        mesh=mesh,
        out_type=jax.ShapeDtypeStruct((total_rows, d), jnp.float32),
        scratch_types=[
            pltpu.VMEM((CHUNK,), jnp.int32),
            pltpu.VMEM((CHUNK,), jnp.int32),
            pltpu.VMEM((CHUNK, d), jnp.float32),
            pltpu.VMEM((CHUNK, d), jnp.float32),
            pltpu.VMEM((CHUNK, d), jnp.float32),
            pltpu.VMEM((CHUNK, d), jnp.float32),
            pltpu.SemaphoreType.DMA,
        ],
    )
    def combine_kernel(acc_hbm, idx_hbm, self_hbm, out_hbm,
                       ibuf, ibufn, g0, g1, sbuf, obuf, sem):
        c = lax.axis_index("c")
        s = lax.axis_index("s")
        wid = s * NC + c
        base = wid * rows_per_tile

        for t in range(n_sub):
            b0 = base + t * CHUNK
            pltpu.sync_copy(idx_hbm.at[pl.ds(b0, CHUNK)], ibuf)
            for k8 in range(CHUNK // LANES):
                sl = pl.ds(k8 * LANES, LANES)
                ibufn[sl] = ibuf[sl] + n_nodes
            cp0 = pltpu.async_copy(acc_hbm.at[ibuf], g0, sem)
            cp1 = pltpu.async_copy(acc_hbm.at[ibufn], g1, sem)
            pltpu.sync_copy(self_hbm.at[pl.ds(b0, CHUNK)], sbuf)
            cp0.wait()
            cp1.wait()

            def srow(k, carry):
                for k8 in range(d8):
                    sl = pl.ds(k8 * LANES, LANES)
                    x = g0[k, sl] + g1[k, sl] + sbuf[k, sl]
                    obuf[k, sl] = 1.0 / (1.0 + jnp.exp(-x))
                return carry

            lax.fori_loop(0, CHUNK, srow, 0)
            pltpu.sync_copy(obuf, out_hbm.at[pl.ds(b0, CHUNK)])

    return combine_kernel


def kernel(embeddings, head_idx, head_e, tail_idx, tail_e, adj_indices,
           adj_values, relation_kernel, self_kernel):
    n_nodes, d = embeddings.shape
    n_rel, _, n_edges = adj_indices.shape
    batch = head_e.shape[0]

    bn = 1000
    temb = pl.pallas_call(
        _temb_body,
        grid=(n_rel, n_nodes // bn),
        in_specs=[
            pl.BlockSpec((bn, d), lambda r, n: (n, 0)),
            pl.BlockSpec((1, d, d), lambda r, n: (r, 0, 0)),
        ],
        out_specs=pl.BlockSpec((1, bn, d), lambda r, n: (r, n, 0)),
        out_shape=jax.ShapeDtypeStruct((n_rel, n_nodes, d), jnp.float32),
    )(embeddings, relation_kernel)
    temb_flat = temb.reshape(n_rel * n_nodes, d)

    x = jnp.stack([head_e, tail_e])
    bm = 1024
    sout = pl.pallas_call(
        _self_body,
        grid=(2, batch // bm),
        in_specs=[
            pl.BlockSpec((1, bm, d), lambda i, m: (i, m, 0)),
            pl.BlockSpec((d, d), lambda i, m: (0, 0)),
        ],
        out_specs=pl.BlockSpec((1, bm, d), lambda i, m: (i, m, 0)),
        out_shape=jax.ShapeDtypeStruct((2, batch, d), jnp.float32),
    )(x, self_kernel)
    self_flat = sout.reshape(2 * batch, d)

    total_edges = n_rel * n_edges
    n_chunks = total_edges // CHUNK
    chunks_per_rel = n_edges // CHUNK
    rows2d = adj_indices[:, 0, :].astype(jnp.int32).reshape(n_chunks, CHUNK)
    cols2d = adj_indices[:, 1, :].astype(jnp.int32).reshape(n_chunks, CHUNK)
    vals2d = adj_values.reshape(n_chunks, CHUNK)

    scatter = _make_scatter_kernel(n_nodes, d, n_chunks, chunks_per_rel)
    acc2 = scatter(rows2d, cols2d, vals2d, temb_flat)
    acc_flat = acc2.reshape(NC * n_nodes, d)

    idx_all = jnp.concatenate([head_idx.astype(jnp.int32),
                               tail_idx.astype(jnp.int32)])
    combine = _make_combine_kernel(n_nodes, d, 2 * batch)
    out_all = combine(acc_flat, idx_all, self_flat)

    return (out_all[:batch], out_all[batch:])
